# baseline (speedup 1.0000x reference)
import jax
import jax.numpy as jnp
from jax import lax
from jax.experimental import pallas as pl
from jax.experimental.pallas import tpu as pltpu
from jax.experimental.pallas import tpu_sc as plsc

_D = 64
_SEQ = 200
_ROPE_BASE = 10000.0
_NC = 2
_NS = 16
_NW = _NC * _NS

_SUB = 128
_K = 4
_CH = _SUB * _K


def _tec_body(idx_hbm, table_hbm, trig_hbm, out_hbm, idx_v, rows_v, trig_v, sem):
    wid = lax.axis_index("s") * _NC + lax.axis_index("c")
    rows_total = idx_hbm.shape[0]
    rpw = rows_total // _NW
    base = wid * rpw
    nchunks = rpw // _CH

    pltpu.sync_copy(trig_hbm, trig_v)

    def chunk_body(c, carry):
        row0 = base + c * _CH
        for j in range(_K):
            pltpu.sync_copy(idx_hbm.at[pl.ds(row0 + j * _SUB, _SUB)],
                            idx_v.at[j])
        copies = [
            pltpu.async_copy(table_hbm.at[idx_v.at[j]],
                             rows_v.at[pl.ds(j * _SUB, _SUB)], sem)
            for j in range(_K)
        ]
        for cp in copies:
            cp.wait()

        def row_body(r, carry2):
            l = lax.rem(c * _CH + r, _SEQ)
            c0 = trig_v[l, pl.ds(0, 16)]
            c1 = trig_v[l, pl.ds(16, 16)]
            s0 = trig_v[l, pl.ds(32, 16)]
            s1 = trig_v[l, pl.ds(48, 16)]
            h0 = rows_v[r, pl.ds(0, 16)]
            h1 = rows_v[r, pl.ds(16, 16)]
            h2 = rows_v[r, pl.ds(32, 16)]
            h3 = rows_v[r, pl.ds(48, 16)]
            rows_v[r, pl.ds(0, 16)] = h0 * c0 - h2 * s0
            rows_v[r, pl.ds(16, 16)] = h1 * c1 - h3 * s1
            rows_v[r, pl.ds(32, 16)] = h2 * c0 + h0 * s0
            rows_v[r, pl.ds(48, 16)] = h3 * c1 + h1 * s1
            return carry2

        lax.fori_loop(0, _CH, row_body, 0)
        pltpu.sync_copy(rows_v, out_hbm.at[pl.ds(row0, _CH)])
        return carry

    lax.fori_loop(0, nchunks, chunk_body, 0)


def kernel(x, emb_table, pos_table):
    del pos_table
    b, l = x.shape
    rows = b * l
    idx = x.reshape(rows).astype(jnp.int32)

    half = _D // 2
    fi = jnp.arange(half, dtype=jnp.float32)
    freqs = 1.0 / (_ROPE_BASE ** (fi / half))
    ang = jnp.arange(l, dtype=jnp.float32)[:, None] * freqs[None, :]
    trig = jnp.concatenate([jnp.cos(ang), jnp.sin(ang)], axis=1)

    mesh = plsc.VectorSubcoreMesh(core_axis_name="c", subcore_axis_name="s")
    out = pl.kernel(
        _tec_body,
        out_type=jax.ShapeDtypeStruct((rows, _D), jnp.float32),
        mesh=mesh,
        compiler_params=pltpu.CompilerParams(use_tc_tiling_on_sc=False),
        scratch_types=[
            pltpu.VMEM((_K, _SUB), jnp.int32),
            pltpu.VMEM((_CH, _D), jnp.float32),
            pltpu.VMEM((_SEQ, _D), jnp.float32),
            pltpu.SemaphoreType.DMA,
        ],
    )(idx, emb_table, trig)
    return out.reshape(b, l, _D)


# speedup vs baseline: 1.2508x; 1.2508x over previous
import jax
import jax.numpy as jnp
from jax import lax
from jax.experimental import pallas as pl
from jax.experimental.pallas import tpu as pltpu
from jax.experimental.pallas import tpu_sc as plsc

_D = 64
_ROPE_BASE = 10000.0
_NC = 2
_NS = 16
_NW = _NC * _NS
_BPW = 128
_PAD = 128


def _tec_body(idx_hbm, table_hbm, trig_hbm, out_hbm,
              idx_v, trig_v, rows0, rows1, ob0, ob1, sg0, sg1, sw0, sw1):
    seq = idx_hbm.shape[1]
    wid = lax.axis_index("s") * _NC + lax.axis_index("c")
    b0 = wid * _BPW

    pltpu.sync_copy(idx_hbm.at[wid], idx_v)
    pltpu.sync_copy(trig_hbm, trig_v)
    tbl = table_hbm

    rows = (rows0, rows1)
    ob = (ob0, ob1)
    sg = (sg0, sg1)
    sw = (sw0, sw1)

    pltpu.async_copy(tbl.at[idx_v.at[0]], rows0, sg0)

    def step(l2, carry):
        for p in range(2):
            l = l2 * 2 + p

            @pl.when(l + 1 < seq)
            def _fire_next():
                pltpu.async_copy(tbl.at[idx_v.at[l + 1]], rows[1 - p],
                                 sg[1 - p])

            pltpu.make_async_copy(tbl.at[idx_v.at[l]], rows[p],
                                  sg[p]).wait()

            @pl.when(l >= 2)
            def _drain_write():
                pltpu.make_async_copy(ob[p], out_hbm.at[pl.ds(b0, _BPW), l],
                                      sw[p]).wait()

            c0 = trig_v[l, pl.ds(0, 16)]
            c1 = trig_v[l, pl.ds(16, 16)]
            s0 = trig_v[l, pl.ds(32, 16)]
            s1 = trig_v[l, pl.ds(48, 16)]
            ns0 = trig_v[l, pl.ds(64, 16)]
            ns1 = trig_v[l, pl.ds(80, 16)]

            rp = rows[p]
            op = ob[p]

            @plsc.parallel_loop(0, _BPW, 1, unroll=8)
            def _rope_row(r):
                h0 = rp[r, pl.ds(0, 16)]
                h1 = rp[r, pl.ds(16, 16)]
                h2 = rp[r, pl.ds(32, 16)]
                h3 = rp[r, pl.ds(48, 16)]
                op[r, pl.ds(0, 16)] = h0 * c0 + h2 * ns0
                op[r, pl.ds(16, 16)] = h1 * c1 + h3 * ns1
                op[r, pl.ds(32, 16)] = h2 * c0 + h0 * s0
                op[r, pl.ds(48, 16)] = h3 * c1 + h1 * s1

            pltpu.async_copy(ob[p], out_hbm.at[pl.ds(b0, _BPW), l], sw[p])
        return carry

    lax.fori_loop(0, seq // 2, step, 0)

    pltpu.make_async_copy(ob[0], out_hbm.at[pl.ds(b0, _BPW), 0], sw[0]).wait()
    pltpu.make_async_copy(ob[1], out_hbm.at[pl.ds(b0, _BPW), 1], sw[1]).wait()


def kernel(x, emb_table, pos_table):
    del pos_table
    b, l = x.shape
    idx = x.reshape(_NW, _BPW, l).transpose(0, 2, 1).astype(jnp.int32)

    half = _D // 2
    fi = jnp.arange(half, dtype=jnp.float32)
    freqs = 1.0 / (_ROPE_BASE ** (fi / half))
    ang = jnp.arange(l, dtype=jnp.float32)[:, None] * freqs[None, :]
    cos, sin = jnp.cos(ang), jnp.sin(ang)
    trig = jnp.concatenate(
        [cos, sin, -sin, jnp.zeros((l, half), jnp.float32)], axis=1)

    mesh = plsc.VectorSubcoreMesh(core_axis_name="c", subcore_axis_name="s")
    out = pl.kernel(
        _tec_body,
        out_type=jax.ShapeDtypeStruct((b, l, _D), jnp.float32),
        mesh=mesh,
        compiler_params=pltpu.CompilerParams(use_tc_tiling_on_sc=False),
        scratch_types=[
            pltpu.VMEM((l, _BPW), jnp.int32),
            pltpu.VMEM((l, _PAD), jnp.float32),
            pltpu.VMEM((_BPW, _D), jnp.float32),
            pltpu.VMEM((_BPW, _D), jnp.float32),
            pltpu.VMEM((_BPW, _D), jnp.float32),
            pltpu.VMEM((_BPW, _D), jnp.float32),
            pltpu.SemaphoreType.DMA,
            pltpu.SemaphoreType.DMA,
            pltpu.SemaphoreType.DMA,
            pltpu.SemaphoreType.DMA,
        ],
    )(idx, emb_table, trig)
    return out
